# Initial kernel scaffold; baseline (speedup 1.0000x reference)
import jax
import jax.numpy as jnp
from jax.experimental import pallas as pl


def kernel(x, pos, edge_index, edge_attr, We1, be1, We2, be2, Wn1, bn1, Wn2, bn2, Wc, bc):
    raise NotImplementedError("write your pallas kernel here")


# speedup vs baseline: 2.4214x; 2.4214x over previous
import functools

import jax
import jax.numpy as jnp
from jax import lax
from jax.experimental import pallas as pl
from jax.experimental.pallas import tpu as pltpu
from jax.experimental.pallas import tpu_sc as plsc

N = 10000
E = 320000
IN_F = 128
HID = 128
EDGE_F = 16

POSW = 16
NP = 10240
EP = 327680
CH = 128
GRP = CH // 16
TILES = 32
EPT = EP // TILES
NCH = EPT // CH
RPT = NP // 16
NA = 10112
RPA = NA // 16
NB = 256
BE = 1024


def _silu(v):
    return v * jax.nn.sigmoid(v)


def _tables_body(x_ref, wr_ref, wc_ref, be1_ref, tr_ref, tc_ref):
    x = x_ref[...]
    tr_ref[...] = (
        jnp.dot(x, wr_ref[...], preferred_element_type=jnp.float32) + be1_ref[...]
    )
    tc_ref[...] = jnp.dot(x, wc_ref[...], preferred_element_type=jnp.float32)


def _build_tables(xp, wr, wc, be1_2d):
    return pl.pallas_call(
        _tables_body,
        grid=(NP // NB,),
        in_specs=[
            pl.BlockSpec((NB, IN_F), lambda i: (i, 0)),
            pl.BlockSpec((IN_F, HID), lambda i: (0, 0)),
            pl.BlockSpec((IN_F, HID), lambda i: (0, 0)),
            pl.BlockSpec((1, HID), lambda i: (0, 0)),
        ],
        out_specs=[
            pl.BlockSpec((NB, HID), lambda i: (i, 0)),
            pl.BlockSpec((NB, HID), lambda i: (i, 0)),
        ],
        out_shape=[jax.ShapeDtypeStruct((NP, HID), jnp.float32)] * 2,
    )(xp, wr, wc, be1_2d)


@functools.cache
def _gather_kernel():
    mesh = plsc.VectorSubcoreMesh(core_axis_name="c", subcore_axis_name="s")

    @functools.partial(
        pl.kernel,
        out_type=[
            jax.ShapeDtypeStruct((EP, HID), jnp.float32),
            jax.ShapeDtypeStruct((EP, HID), jnp.float32),
            jax.ShapeDtypeStruct((EP, POSW), jnp.float32),
        ],
        mesh=mesh,
        compiler_params=pltpu.CompilerParams(needs_layout_passes=False),
        scratch_types=[
            pltpu.VMEM((CH,), jnp.int32),
            pltpu.VMEM((CH,), jnp.int32),
            pltpu.VMEM((CH, HID), jnp.float32),
            pltpu.VMEM((CH, HID), jnp.float32),
            pltpu.VMEM((CH, POSW), jnp.float32),
            pltpu.VMEM((NP,), jnp.float32),
            pltpu.VMEM((NP,), jnp.float32),
            pltpu.VMEM((NP,), jnp.float32),
            pltpu.SemaphoreType.DMA,
            pltpu.SemaphoreType.DMA,
        ],
    )
    def _gather_k(tr_hbm, tc_hbm, px_hbm, py_hbm, pz_hbm, row_hbm, col_hbm,
                  xr_hbm, xc_hbm, rd_hbm,
                  idx_r, idx_c, buf_r, buf_c, rdbuf, pxv, pyv, pzv,
                  sem_r, sem_c):
        c = lax.axis_index("c")
        s = lax.axis_index("s")
        base = (s * 2 + c) * EPT

        pltpu.sync_copy(px_hbm, pxv)
        pltpu.sync_copy(py_hbm, pyv)
        pltpu.sync_copy(pz_hbm, pzv)

        def clr(i, carry):
            rdbuf[i, :] = jnp.zeros((POSW,), jnp.float32)
            return carry
        lax.fori_loop(0, CH, clr, 0)

        def body(j, carry):
            ofs = base + j * CH
            pltpu.sync_copy(row_hbm.at[pl.ds(ofs, CH)], idx_r)
            pltpu.sync_copy(col_hbm.at[pl.ds(ofs, CH)], idx_c)
            cp_r = pltpu.async_copy(tr_hbm.at[idx_r], buf_r, sem_r)
            cp_c = pltpu.async_copy(tc_hbm.at[idx_c], buf_c, sem_c)
            for g in range(GRP):
                ivr = idx_r[pl.ds(g * 16, 16)]
                ivc = idx_c[pl.ds(g * 16, 16)]
                rx = plsc.load_gather(pxv, [ivr]) - plsc.load_gather(pxv, [ivc])
                ry = plsc.load_gather(pyv, [ivr]) - plsc.load_gather(pyv, [ivc])
                rz = plsc.load_gather(pzv, [ivr]) - plsc.load_gather(pzv, [ivc])
                dij = rx * rx + ry * ry + rz * rz
                rows = lax.iota(jnp.int32, 16) + (g * 16)
                plsc.store_scatter(rdbuf, [rows, jnp.full((16,), 0, jnp.int32)], rx)
                plsc.store_scatter(rdbuf, [rows, jnp.full((16,), 1, jnp.int32)], ry)
                plsc.store_scatter(rdbuf, [rows, jnp.full((16,), 2, jnp.int32)], rz)
                plsc.store_scatter(rdbuf, [rows, jnp.full((16,), 3, jnp.int32)], dij)
            cp_r.wait()
            cp_c.wait()
            pltpu.sync_copy(buf_r, xr_hbm.at[pl.ds(ofs, CH)])
            pltpu.sync_copy(buf_c, xc_hbm.at[pl.ds(ofs, CH)])
            pltpu.sync_copy(rdbuf, rd_hbm.at[pl.ds(ofs, CH)])
            return carry

        lax.fori_loop(0, NCH, body, 0)

    return _gather_k


def _edge_body(xr_ref, xc_ref, rd_ref, ea_ref, we2_ref, we1e_ref, sw_ref,
               m_ref, tr_ref):
    rd = rd_ref[...]
    dij = rd[:, 3:4]
    wd = sw_ref[0:1, :]
    be2 = sw_ref[1:2, :]
    wc_row = sw_ref[2:3, :]
    bc = sw_ref[3:4, 0:1]
    h1 = xr_ref[...] + xc_ref[...] + dij * wd + jnp.dot(
        ea_ref[...], we1e_ref[...], preferred_element_type=jnp.float32)
    h1 = _silu(h1)
    m = jnp.dot(h1, we2_ref[...], preferred_element_type=jnp.float32) + be2
    m = _silu(m)
    wij = _silu(jnp.sum(m * wc_row, axis=1, keepdims=True) + bc)
    lane = lax.broadcasted_iota(jnp.int32, rd.shape, 1)
    ridx = (lax.broadcasted_iota(jnp.int32, (m.shape[0], 1), 0)
            + pl.program_id(0) * BE)
    valid = ridx < E
    m_ref[...] = jnp.where(valid, m, 0.0)
    tr_ref[...] = jnp.where(valid & (lane < 3), rd * wij, 0.0)


def _edge_mlp(xr, xc, rd, eap, we2, we1e, sw):
    return pl.pallas_call(
        _edge_body,
        grid=(EP // BE,),
        in_specs=[
            pl.BlockSpec((BE, HID), lambda i: (i, 0)),
            pl.BlockSpec((BE, HID), lambda i: (i, 0)),
            pl.BlockSpec((BE, POSW), lambda i: (i, 0)),
            pl.BlockSpec((BE, EDGE_F), lambda i: (i, 0)),
            pl.BlockSpec((HID, HID), lambda i: (0, 0)),
            pl.BlockSpec((EDGE_F, HID), lambda i: (0, 0)),
            pl.BlockSpec((8, HID), lambda i: (0, 0)),
        ],
        out_specs=[
            pl.BlockSpec((BE, HID), lambda i: (i, 0)),
            pl.BlockSpec((BE, POSW), lambda i: (i, 0)),
        ],
        out_shape=[
            jax.ShapeDtypeStruct((EP, HID), jnp.float32),
            jax.ShapeDtypeStruct((EP, POSW), jnp.float32),
        ],
    )(xr, xc, rd, eap, we2, we1e, sw)


@functools.cache
def _scatter_kernel():
    mesh = plsc.VectorSubcoreMesh(core_axis_name="c", subcore_axis_name="s")

    @functools.partial(
        pl.kernel,
        out_type=[jax.ShapeDtypeStruct((2 * NP, HID), jnp.float32)],
        mesh=mesh,
        compiler_params=pltpu.CompilerParams(needs_layout_passes=False),
        scratch_types=[
            pltpu.VMEM((CH,), jnp.int32),
            pltpu.VMEM((CH, HID), jnp.float32),
            pltpu.VMEM_SHARED((NA, HID), jnp.float32),
        ],
    )
    def _scatter_k(m_hbm, row_hbm, zg_hbm, agg_hbm, idx, mbuf, acc_g):
        c = lax.axis_index("c")
        s = lax.axis_index("s")
        r0 = s * RPA

        def zinit(k, carry):
            rr = r0 + k * CH
            pltpu.sync_copy(zg_hbm.at[pl.ds(rr, CH)], mbuf)
            pltpu.sync_copy(mbuf, acc_g.at[pl.ds(rr, CH)])
            return carry

        lax.fori_loop(0, 4, zinit, 0)
        rt = r0 + 4 * CH
        pltpu.sync_copy(zg_hbm.at[pl.ds(rt, RPA - 4 * CH)],
                        mbuf.at[pl.ds(0, RPA - 4 * CH)])
        pltpu.sync_copy(mbuf.at[pl.ds(0, RPA - 4 * CH)],
                        acc_g.at[pl.ds(rt, RPA - 4 * CH)])
        plsc.subcore_barrier()
        base = c * (EP // 2) + s * EPT

        def body(j, carry):
            ofs = base + j * CH
            pltpu.sync_copy(row_hbm.at[pl.ds(ofs, CH)], idx)
            pltpu.sync_copy(m_hbm.at[pl.ds(ofs, CH)], mbuf)
            pltpu.sync_copy(mbuf, acc_g.at[idx], add=True)
            return carry

        lax.fori_loop(0, NCH, body, 0)
        plsc.subcore_barrier()

        def wout(k, carry):
            rr = r0 + k * CH
            pltpu.sync_copy(acc_g.at[pl.ds(rr, CH)], mbuf)
            pltpu.sync_copy(mbuf, agg_hbm.at[pl.ds(c * NP + rr, CH)])
            return carry

        lax.fori_loop(0, 4, wout, 0)
        rt2 = r0 + 4 * CH
        pltpu.sync_copy(acc_g.at[pl.ds(rt2, RPA - 4 * CH)],
                        mbuf.at[pl.ds(0, RPA - 4 * CH)])
        pltpu.sync_copy(mbuf.at[pl.ds(0, RPA - 4 * CH)],
                        agg_hbm.at[pl.ds(c * NP + rt2, RPA - 4 * CH)])

        @pl.when(s == 0)
        def _tail():
            pltpu.sync_copy(zg_hbm.at[pl.ds(0, NP - NA)], mbuf)
            pltpu.sync_copy(mbuf, agg_hbm.at[pl.ds(c * NP + NA, NP - NA)])

    return _scatter_k


NDW = NP // 8
NRW = NDW // 16


@functools.cache
def _delta_kernel():
    mesh = plsc.VectorSubcoreMesh(core_axis_name="c", subcore_axis_name="s")

    @functools.partial(
        pl.kernel,
        out_type=[jax.ShapeDtypeStruct((2 * NDW, HID), jnp.float32)],
        mesh=mesh,
        compiler_params=pltpu.CompilerParams(needs_layout_passes=False),
        scratch_types=[
            pltpu.VMEM((CH,), jnp.int32),
            pltpu.VMEM((CH,), jnp.int32),
            pltpu.VMEM((CH, POSW), jnp.float32),
            pltpu.VMEM((CH, HID), jnp.float32),
            pltpu.VMEM((NRW, HID), jnp.float32),
            pltpu.VMEM_SHARED((NDW, HID), jnp.float32),
        ],
    )
    def _delta_k(trn_hbm, row_hbm, zg_hbm, dlw_hbm,
                 idx, idx8, tbuf, tbufw, wbuf, acc_dw):
        c = lax.axis_index("c")
        s = lax.axis_index("s")
        rw0 = s * NRW

        def zwide(i, carry):
            for t in range(8):
                tbufw[i, pl.ds(t * 16, 16)] = jnp.zeros((16,), jnp.float32)
            return carry

        lax.fori_loop(0, CH, zwide, 0)
        pltpu.sync_copy(zg_hbm.at[pl.ds(0, NRW)], wbuf)
        pltpu.sync_copy(wbuf, acc_dw.at[pl.ds(rw0, NRW)])
        plsc.subcore_barrier()
        base = c * (EP // 2) + s * EPT

        def body(j, carry):
            ofs = base + j * CH
            pltpu.sync_copy(row_hbm.at[pl.ds(ofs, CH)], idx)
            pltpu.sync_copy(trn_hbm.at[pl.ds(ofs, CH)], tbuf)
            for g in range(GRP):
                il = lax.iota(jnp.int32, 16) + (g * 16)
                iv = idx[pl.ds(g * 16, 16)]
                lb = (iv & 7) * 16
                idx8[pl.ds(g * 16, 16)] = lax.shift_right_logical(iv, 3)
                tx = plsc.load_gather(tbuf, [il, jnp.full((16,), 0, jnp.int32)])
                ty = plsc.load_gather(tbuf, [il, jnp.full((16,), 1, jnp.int32)])
                tz = plsc.load_gather(tbuf, [il, jnp.full((16,), 2, jnp.int32)])
                plsc.store_scatter(tbufw, [il, lb], tx)
                plsc.store_scatter(tbufw, [il, lb + 1], ty)
                plsc.store_scatter(tbufw, [il, lb + 2], tz)
            pltpu.sync_copy(tbufw, acc_dw.at[idx8], add=True)
            zero16 = jnp.zeros((16,), jnp.float32)
            for g in range(GRP):
                il = lax.iota(jnp.int32, 16) + (g * 16)
                lb = (idx[pl.ds(g * 16, 16)] & 7) * 16
                plsc.store_scatter(tbufw, [il, lb], zero16)
                plsc.store_scatter(tbufw, [il, lb + 1], zero16)
                plsc.store_scatter(tbufw, [il, lb + 2], zero16)
            return carry

        lax.fori_loop(0, NCH, body, 0)
        plsc.subcore_barrier()
        pltpu.sync_copy(acc_dw.at[pl.ds(rw0, NRW)], wbuf)
        pltpu.sync_copy(wbuf, dlw_hbm.at[pl.ds(c * NDW + rw0, NRW)])

    return _delta_k


def _node_body(x_ref, pos_ref, agg_ref, dl_ref, wn1x_ref, wn1a_ref, wn2_ref,
               b_ref, xn_ref, pn_ref):
    x = x_ref[...]
    agg = agg_ref[0] + agg_ref[1]
    bn1 = b_ref[0:1, :]
    bn2 = b_ref[1:2, :]
    h2 = (jnp.dot(x, wn1x_ref[...], preferred_element_type=jnp.float32)
          + jnp.dot(agg, wn1a_ref[...], preferred_element_type=jnp.float32)
          + bn1)
    h2 = _silu(h2)
    xn_ref[...] = jnp.dot(h2, wn2_ref[...], preferred_element_type=jnp.float32) + bn2
    pn_ref[...] = pos_ref[...] + 0.01 * (dl_ref[0] + dl_ref[1])


def _node_mlp(xp, pos16, aggp, dlp, wn1x, wn1a, wn2, b2):
    return pl.pallas_call(
        _node_body,
        grid=(NP // NB,),
        in_specs=[
            pl.BlockSpec((NB, IN_F), lambda i: (i, 0)),
            pl.BlockSpec((NB, POSW), lambda i: (i, 0)),
            pl.BlockSpec((2, NB, HID), lambda i: (0, i, 0)),
            pl.BlockSpec((2, NB, POSW), lambda i: (0, i, 0)),
            pl.BlockSpec((IN_F, HID), lambda i: (0, 0)),
            pl.BlockSpec((HID, HID), lambda i: (0, 0)),
            pl.BlockSpec((HID, IN_F), lambda i: (0, 0)),
            pl.BlockSpec((8, HID), lambda i: (0, 0)),
        ],
        out_specs=[
            pl.BlockSpec((NB, IN_F), lambda i: (i, 0)),
            pl.BlockSpec((NB, POSW), lambda i: (i, 0)),
        ],
        out_shape=[
            jax.ShapeDtypeStruct((NP, IN_F), jnp.float32),
            jax.ShapeDtypeStruct((NP, POSW), jnp.float32),
        ],
    )(xp, pos16, aggp, dlp, wn1x, wn1a, wn2, b2)


def kernel(x, pos, edge_index, edge_attr, We1, be1, We2, be2,
           Wn1, bn1, Wn2, bn2, Wc, bc):
    f32 = jnp.float32
    ei = edge_index.astype(jnp.int32)
    pad_idx = jnp.zeros((EP - E,), jnp.int32)
    row = jnp.concatenate([ei[0], pad_idx])
    col = jnp.concatenate([ei[1], pad_idx])
    xp = jnp.pad(x, ((0, NP - N), (0, 0)))
    posp = jnp.pad(pos, ((0, NP - N), (0, 0)))
    px, py, pz = posp[:, 0], posp[:, 1], posp[:, 2]
    pos16 = jnp.pad(posp, ((0, 0), (0, POSW - 3)))
    eap = jnp.pad(edge_attr, ((0, EP - E), (0, 0)))

    wr = We1[:IN_F]
    wc = We1[IN_F:2 * IN_F]
    sw = (jnp.zeros((8, HID), f32)
          .at[0].set(We1[2 * IN_F])
          .at[1].set(be2)
          .at[2].set(Wc[:, 0])
          .at[3].set(jnp.full((HID,), bc[0], f32)))
    we1e = We1[2 * IN_F + 1:]
    b2 = jnp.zeros((8, HID), f32).at[0].set(bn1).at[1].set(bn2)
    zg = jnp.zeros((NP, HID), f32)

    t_r, t_c = _build_tables(xp, wr, wc, be1.reshape(1, HID))
    xr, xc_g, rd = _gather_kernel()(t_r, t_c, px, py, pz, row, col)
    m, trn = _edge_mlp(xr, xc_g, rd, eap, We2, we1e, sw)
    (aggp,) = _scatter_kernel()(m, row, zg)
    (dlw,) = _delta_kernel()(trn, row, zg)
    aggp = aggp.reshape(2, NP, HID)
    dlp = dlw.reshape(2, NP, POSW)
    xn, pn = _node_mlp(xp, pos16, aggp, dlp, Wn1[:IN_F], Wn1[IN_F:], Wn2, b2)
    return (xn[:N], pn[:N, :3])
